# baseline (speedup 1.0000x reference)
import functools

import jax
import jax.numpy as jnp
from jax import lax
from jax.experimental import pallas as pl
from jax.experimental.pallas import tpu as pltpu
from jax.experimental.pallas import tpu_sc as plsc

_NC = 2
_NS = 16
_NW = _NC * _NS
_CHUNK = 128


@functools.lru_cache(maxsize=None)
def _make_count(k: int, n_cnt: int):
    mesh = plsc.VectorSubcoreMesh(core_axis_name="c", subcore_axis_name="s")
    tile_rows = n_cnt // _NS

    @functools.partial(
        pl.kernel,
        out_type=jax.ShapeDtypeStruct((_NC * n_cnt,), jnp.float32),
        mesh=mesh,
        scratch_types=[
            pltpu.VMEM((k, _CHUNK), jnp.int32),
            pltpu.VMEM((_CHUNK,), jnp.float32),
            pltpu.VMEM((tile_rows,), jnp.float32),
            pltpu.VMEM_SHARED((n_cnt,), jnp.float32),
            pltpu.SemaphoreType.DMA,
        ],
    )
    def cnt(edges_hbm, out_hbm, didx, ones, zeros, acc, sem):
        c = lax.axis_index("c")
        s = lax.axis_index("s")
        wid = s * _NC + c
        cp = pltpu.async_copy(edges_hbm.at[1, wid], didx, sem)

        onev = jnp.ones((16,), jnp.float32)
        zv = jnp.zeros((16,), jnp.float32)
        for j in range(_CHUNK // 16):
            ones[pl.ds(j * 16, 16)] = onev

        def zfill(i, _):
            zeros[pl.ds(i * 16, 16)] = zv
            return 0

        lax.fori_loop(0, tile_rows // 16, zfill, 0)
        pltpu.sync_copy(zeros, acc.at[pl.ds(s * tile_rows, tile_rows)])
        cp.wait()
        plsc.subcore_barrier()

        def body(j, _):
            pltpu.sync_copy(ones, acc.at[didx.at[j]], add=True)
            return 0

        lax.fori_loop(0, k, body, 0)
        plsc.subcore_barrier()
        pltpu.sync_copy(acc.at[pl.ds(s * tile_rows, tile_rows)],
                        out_hbm.at[pl.ds(c * n_cnt + s * tile_rows, tile_rows)])

    return cnt


@functools.lru_cache(maxsize=None)
def _make_segsum(n: int, d: int, k: int):
    n_out = -(-n // (_NS * 8)) * _NS * 8
    rpt = n_out // _NS
    kb = 16
    assert k % kb == 0
    deep = d <= 32
    mesh = plsc.VectorSubcoreMesh(core_axis_name="c", subcore_axis_name="s")

    if deep:
        return _make_segsum_deep(n, d, k, n_out, rpt, mesh)

    @functools.partial(
        pl.kernel,
        out_type=jax.ShapeDtypeStruct((_NC, n_out, d), jnp.float32),
        mesh=mesh,
        compiler_params=pltpu.CompilerParams(use_tc_tiling_on_sc=(d % 128 == 0)),
        scratch_types=[
            pltpu.VMEM((2, kb, _CHUNK), jnp.int32),
            pltpu.VMEM((2, kb, _CHUNK), jnp.int32),
            pltpu.VMEM((_CHUNK, d), jnp.float32),
            pltpu.VMEM((_CHUNK, d), jnp.float32),
            pltpu.VMEM_SHARED((n_out, d), jnp.float32),
            pltpu.SemaphoreType.DMA,
            pltpu.SemaphoreType.DMA,
            pltpu.SemaphoreType.DMA,
            pltpu.SemaphoreType.DMA,
            pltpu.SemaphoreType.DMA,
            pltpu.SemaphoreType.DMA,
        ],
    )
    def segsum(hw_hbm, edges_hbm, out_hbm,
               sidx, didx, bufa, bufb, acc, sga, sgb, ssa, ssb, six0, six1):
        c = lax.axis_index("c")
        s = lax.axis_index("s")
        wid = s * _NC + c
        sixs = [six0, six1]

        pltpu.async_copy(edges_hbm.at[0, wid, pl.ds(0, kb)], sidx.at[0], six0)
        pltpu.async_copy(edges_hbm.at[1, wid, pl.ds(0, kb)], didx.at[0], six0)

        zv = jnp.zeros((16,), jnp.float32)

        def zrow(i, _):
            for j in range(d // 16):
                bufa[i, pl.ds(j * 16, 16)] = zv
            return 0

        lax.fori_loop(0, _CHUNK, zrow, 0)

        base = s * rpt
        nfull, rem = divmod(rpt, _CHUNK)
        for j in range(nfull):
            pltpu.sync_copy(bufa, acc.at[pl.ds(base + j * _CHUNK, _CHUNK), :])
        if rem:
            pltpu.sync_copy(bufa.at[pl.ds(0, rem), :],
                            acc.at[pl.ds(base + nfull * _CHUNK, rem), :])
        plsc.subcore_barrier()

        nb = k // kb
        for b in range(nb):
            sl = b % 2
            if b + 1 < nb:
                pltpu.async_copy(edges_hbm.at[0, wid, pl.ds((b + 1) * kb, kb)],
                                 sidx.at[1 - sl], sixs[1 - sl])
                pltpu.async_copy(edges_hbm.at[1, wid, pl.ds((b + 1) * kb, kb)],
                                 didx.at[1 - sl], sixs[1 - sl])
            pltpu.make_async_copy(edges_hbm.at[0, wid, pl.ds(0, kb)],
                                  sidx.at[sl], sixs[sl]).wait()
            pltpu.make_async_copy(edges_hbm.at[1, wid, pl.ds(0, kb)],
                                  didx.at[sl], sixs[sl]).wait()

            def gstart(j, buf, sem):
                return pltpu.async_copy(hw_hbm.at[sidx.at[sl, j]], buf, sem)

            def sstart(j, buf, sem):
                return pltpu.async_copy(buf, acc.at[didx.at[sl, j]], sem,
                                        add=True)

            ga = gstart(0, bufa, sga)
            gb = gstart(1, bufb, sgb)

            def body(p, _):
                j = 2 * p
                ga.wait()
                sa = sstart(j, bufa, ssa)
                gb.wait()
                sb = sstart(j + 1, bufb, ssb)
                sa.wait()
                gstart(j + 2, bufa, sga)
                sb.wait()
                gstart(j + 3, bufb, sgb)
                return 0

            lax.fori_loop(0, kb // 2 - 2, body, 0)
            ga.wait()
            sa = sstart(kb - 4, bufa, ssa)
            gb.wait()
            sb = sstart(kb - 3, bufb, ssb)
            sa.wait()
            gstart(kb - 2, bufa, sga)
            sb.wait()
            gstart(kb - 1, bufb, sgb)
            ga.wait()
            sa = sstart(kb - 2, bufa, ssa)
            gb.wait()
            sb = sstart(kb - 1, bufb, ssb)
            sa.wait()
            sb.wait()

        plsc.subcore_barrier()
        pltpu.sync_copy(acc.at[pl.ds(s * rpt, rpt), :],
                        out_hbm.at[c, pl.ds(s * rpt, rpt), :])

    return segsum


def _make_segsum_deep(n: int, d: int, k: int, n_out: int, rpt: int, mesh):
    NB = 4
    assert k % (2 * NB) == 0 and k >= 4 * NB
    R = k // (2 * NB)

    @functools.partial(
        pl.kernel,
        out_type=jax.ShapeDtypeStruct((_NC, n_out, d), jnp.float32),
        mesh=mesh,
        compiler_params=pltpu.CompilerParams(use_tc_tiling_on_sc=(d % 128 == 0)),
        scratch_types=[
            pltpu.VMEM((k, _CHUNK), jnp.int32),
            pltpu.VMEM((k, _CHUNK), jnp.int32),
        ] + [pltpu.VMEM((_CHUNK, d), jnp.float32) for _ in range(2 * NB)] + [
            pltpu.VMEM_SHARED((n_out, d), jnp.float32),
            pltpu.SemaphoreType.DMA,
            pltpu.SemaphoreType.DMA,
            pltpu.SemaphoreType.DMA,
            pltpu.SemaphoreType.DMA,
        ],
    )
    def segsum(hw_hbm, edges_hbm, out_hbm, sidx, didx,
               a0, a1, a2, a3, b0, b1, b2, b3, acc, gsa, gsb, ssa, ssb):
        c = lax.axis_index("c")
        s = lax.axis_index("s")
        wid = s * _NC + c
        bufa = [a0, a1, a2, a3]
        bufb = [b0, b1, b2, b3]

        zv = jnp.zeros((16,), jnp.float32)

        def zrow(i, _):
            for j in range(d // 16):
                a0[i, pl.ds(j * 16, 16)] = zv
            return 0

        lax.fori_loop(0, _CHUNK, zrow, 0)
        zbase = s * rpt
        nfull, rem = divmod(rpt, _CHUNK)
        for j in range(nfull):
            pltpu.sync_copy(a0, acc.at[pl.ds(zbase + j * _CHUNK, _CHUNK), :])
        if rem:
            pltpu.sync_copy(a0.at[pl.ds(0, rem), :],
                            acc.at[pl.ds(zbase + nfull * _CHUNK, rem), :])

        pltpu.sync_copy(edges_hbm.at[0, wid], sidx)
        pltpu.sync_copy(edges_hbm.at[1, wid], didx)
        plsc.subcore_barrier()

        def fire_g(j, buf, sem):
            return pltpu.async_copy(hw_hbm.at[sidx.at[j]], buf, sem)

        def fire_s(j, buf, sem):
            return pltpu.async_copy(buf, acc.at[didx.at[j]], sem, add=True)

        def half_round(base, bufs, gsem, ssem, nxt):
            hs = []
            for i in range(NB):
                pltpu.make_async_copy(hw_hbm.at[sidx.at[base + i]],
                                      bufs[i], gsem).wait()
                hs.append(fire_s(base + i, bufs[i], ssem))
            for h in hs:
                h.wait()
            if nxt is not None:
                for i in range(NB):
                    fire_g(nxt + i, bufs[i], gsem)

        for i in range(NB):
            fire_g(i, bufa[i], gsa)

        def round_body(r, _):
            base = 2 * NB * r
            for i in range(NB):
                fire_g(base + NB + i, bufb[i], gsb)
            half_round(base, bufa, gsa, ssa, base + 2 * NB)
            half_round(base + NB, bufb, gsb, ssb, None)
            return 0

        lax.fori_loop(0, R - 1, round_body, 0)
        base = 2 * NB * (R - 1)
        for i in range(NB):
            fire_g(base + NB + i, bufb[i], gsb)
        half_round(base, bufa, gsa, ssa, None)
        half_round(base + NB, bufb, gsb, ssb, None)

        plsc.subcore_barrier()
        pltpu.sync_copy(acc.at[pl.ds(s * rpt, rpt), :],
                        out_hbm.at[c, pl.ds(s * rpt, rpt), :])

    return segsum


def _first_body(x_ref, w_ref, c0_ref, c1_ref, hw_ref, dis_ref):
    dis = lax.rsqrt(1.0 + c0_ref[...] + c1_ref[...])
    dis_ref[...] = dis
    hw_ref[...] = jnp.dot(x_ref[...], w_ref[...],
                          preferred_element_type=jnp.float32) * dis


def _first_call(x, wt, c0, c1):
    n = x.shape[0]
    return pl.pallas_call(
        _first_body,
        out_shape=(jax.ShapeDtypeStruct((n, wt.shape[1]), jnp.float32),
                   jax.ShapeDtypeStruct((n, 1), jnp.float32)),
    )(x, wt, c0, c1)


def _mid_body(acc_ref, hwp_ref, dis_ref, b_ref, w_ref, out_ref):
    n = hwp_ref.shape[0]
    dis = dis_ref[...]
    h = (acc_ref[0, :n, :] + acc_ref[1, :n, :] + hwp_ref[...]) * dis + b_ref[...]
    h = jnp.maximum(h, 0.0)
    out_ref[...] = jnp.dot(h, w_ref[...],
                           preferred_element_type=jnp.float32) * dis


def _mid_call(acc, hwp, dis, b, wt):
    n = hwp.shape[0]
    return pl.pallas_call(
        _mid_body,
        out_shape=jax.ShapeDtypeStruct((n, wt.shape[1]), jnp.float32),
    )(acc, hwp, dis, b, wt)


def _fin_body(acc_ref, hwp_ref, dis_ref, b_ref, out_ref):
    n = hwp_ref.shape[0]
    out_ref[...] = (acc_ref[0, :n, :] + acc_ref[1, :n, :] + hwp_ref[...]) \
        * dis_ref[...] + b_ref[...]


def _fin_call(acc, hwp, dis, b):
    return pl.pallas_call(
        _fin_body,
        out_shape=jax.ShapeDtypeStruct(hwp.shape, jnp.float32),
    )(acc, hwp, dis, b)


def kernel(x, edge_index, W1, b1, W2, b2, W3, b3):
    n = x.shape[0]
    e = edge_index.shape[1]
    per = _NW * _CHUNK
    k = -(-(-(-e // per)) // 16) * 16
    pad = k * per - e

    pidx = jnp.arange(pad, dtype=jnp.int32)
    pad2d = jnp.stack([pidx % jnp.int32(min(_CHUNK, n)),
                       n + (pidx % jnp.int32(16))])
    et = jnp.concatenate([edge_index, pad2d], axis=1).reshape(2, _NW, k, _CHUNK)

    tile_rows = -(-(n + 16) // (_NS * 128)) * 128
    n_cnt = tile_rows * _NS

    counts = _make_count(k, n_cnt)(et)
    c0 = counts[:n].reshape(n, 1)
    c1 = counts[n_cnt:n_cnt + n].reshape(n, 1)

    hw1, dis = _first_call(x, W1.T, c0, c1)
    acc1 = _make_segsum(n, W1.shape[0], k)(hw1, et)
    hw2 = _mid_call(acc1, hw1, dis, b1.reshape(1, -1), W2.T)
    acc2 = _make_segsum(n, W2.shape[0], k)(hw2, et)
    hw3 = _mid_call(acc2, hw2, dis, b2.reshape(1, -1), W3.T)
    acc3 = _make_segsum(n, W3.shape[0], k)(hw3, et)
    z = _fin_call(acc3, hw3, dis, b3.reshape(1, -1))
    return z


# speedup vs baseline: 1.1987x; 1.1987x over previous
import functools

import jax
import jax.numpy as jnp
from jax import lax
from jax.experimental import pallas as pl
from jax.experimental.pallas import tpu as pltpu
from jax.experimental.pallas import tpu_sc as plsc

_NC = 2
_NS = 16
_NW = _NC * _NS
_CHUNK = 128


@functools.lru_cache(maxsize=None)
def _make_count(k: int, n_cnt: int):
    mesh = plsc.VectorSubcoreMesh(core_axis_name="c", subcore_axis_name="s")
    tile_rows = n_cnt // _NS

    @functools.partial(
        pl.kernel,
        out_type=jax.ShapeDtypeStruct((_NC * n_cnt,), jnp.float32),
        mesh=mesh,
        scratch_types=[
            pltpu.VMEM((k, _CHUNK), jnp.int32),
            pltpu.VMEM((_CHUNK,), jnp.float32),
            pltpu.VMEM((tile_rows,), jnp.float32),
            pltpu.VMEM_SHARED((n_cnt,), jnp.float32),
            pltpu.SemaphoreType.DMA,
        ],
    )
    def cnt(edges_hbm, out_hbm, didx, ones, zeros, acc, sem):
        c = lax.axis_index("c")
        s = lax.axis_index("s")
        wid = s * _NC + c
        cp = pltpu.async_copy(edges_hbm.at[1, wid], didx, sem)

        onev = jnp.ones((16,), jnp.float32)
        zv = jnp.zeros((16,), jnp.float32)
        for j in range(_CHUNK // 16):
            ones[pl.ds(j * 16, 16)] = onev

        def zfill(i, _):
            zeros[pl.ds(i * 16, 16)] = zv
            return 0

        lax.fori_loop(0, tile_rows // 16, zfill, 0)
        pltpu.sync_copy(zeros, acc.at[pl.ds(s * tile_rows, tile_rows)])
        cp.wait()
        plsc.subcore_barrier()

        def body(j, _):
            pltpu.sync_copy(ones, acc.at[didx.at[j]], add=True)
            return 0

        lax.fori_loop(0, k, body, 0)
        plsc.subcore_barrier()
        pltpu.sync_copy(acc.at[pl.ds(s * tile_rows, tile_rows)],
                        out_hbm.at[pl.ds(c * n_cnt + s * tile_rows, tile_rows)])

    return cnt


@functools.lru_cache(maxsize=None)
def _make_segsum(n: int, d: int, k: int):
    n_out = -(-n // (_NS * 8)) * _NS * 8
    rpt = n_out // _NS
    kb = 40
    assert k % kb == 0
    deep = d <= 32
    mesh = plsc.VectorSubcoreMesh(core_axis_name="c", subcore_axis_name="s")

    if deep:
        return _make_segsum_deep(n, d, k, n_out, rpt, mesh)

    @functools.partial(
        pl.kernel,
        out_type=jax.ShapeDtypeStruct((_NC, n_out, d), jnp.float32),
        mesh=mesh,
        compiler_params=pltpu.CompilerParams(use_tc_tiling_on_sc=(d % 128 == 0)),
        scratch_types=[
            pltpu.VMEM((kb, _CHUNK), jnp.int32),
            pltpu.VMEM((kb, _CHUNK), jnp.int32),
            pltpu.VMEM((_CHUNK, d), jnp.float32),
            pltpu.VMEM((_CHUNK, d), jnp.float32),
            pltpu.VMEM_SHARED((n_out, d), jnp.float32),
            pltpu.SemaphoreType.DMA,
            pltpu.SemaphoreType.DMA,
        ],
    )
    def segsum(hw_hbm, edges_hbm, out_hbm,
               sidx, didx, bufa, bufb, acc, sga, sgb):
        c = lax.axis_index("c")
        s = lax.axis_index("s")
        wid = s * _NC + c

        zv = jnp.zeros((16,), jnp.float32)

        def zrow(i, _):
            for j in range(d // 16):
                bufa[i, pl.ds(j * 16, 16)] = zv
            return 0

        lax.fori_loop(0, _CHUNK, zrow, 0)

        base = s * rpt
        nfull, rem = divmod(rpt, _CHUNK)
        for j in range(nfull):
            pltpu.sync_copy(bufa, acc.at[pl.ds(base + j * _CHUNK, _CHUNK), :])
        if rem:
            pltpu.sync_copy(bufa.at[pl.ds(0, rem), :],
                            acc.at[pl.ds(base + nfull * _CHUNK, rem), :])
        plsc.subcore_barrier()

        def gstart(j, buf, sem):
            return pltpu.async_copy(hw_hbm.at[sidx.at[j]], buf, sem)

        def scat(j, buf):
            pltpu.sync_copy(buf, acc.at[didx.at[j]], add=True)

        for b in range(k // kb):
            pltpu.sync_copy(edges_hbm.at[0, wid, pl.ds(b * kb, kb)], sidx)
            pltpu.sync_copy(edges_hbm.at[1, wid, pl.ds(b * kb, kb)], didx)

            ga = gstart(0, bufa, sga)

            def body(p, _):
                j = 2 * p
                gb = gstart(j + 1, bufb, sgb)
                ga.wait()
                scat(j, bufa)
                gstart(j + 2, bufa, sga)
                gb.wait()
                scat(j + 1, bufb)
                return 0

            lax.fori_loop(0, kb // 2 - 1, body, 0)
            gb = gstart(kb - 1, bufb, sgb)
            ga.wait()
            scat(kb - 2, bufa)
            gb.wait()
            scat(kb - 1, bufb)

        plsc.subcore_barrier()
        pltpu.sync_copy(acc.at[pl.ds(s * rpt, rpt), :],
                        out_hbm.at[c, pl.ds(s * rpt, rpt), :])

    return segsum


def _make_segsum_deep(n: int, d: int, k: int, n_out: int, rpt: int, mesh):
    NB = 4
    assert k % (2 * NB) == 0 and k >= 4 * NB
    R = k // (2 * NB)

    @functools.partial(
        pl.kernel,
        out_type=jax.ShapeDtypeStruct((_NC, n_out, d), jnp.float32),
        mesh=mesh,
        compiler_params=pltpu.CompilerParams(use_tc_tiling_on_sc=(d % 128 == 0)),
        scratch_types=[
            pltpu.VMEM((k, _CHUNK), jnp.int32),
            pltpu.VMEM((k, _CHUNK), jnp.int32),
        ] + [pltpu.VMEM((_CHUNK, d), jnp.float32) for _ in range(2 * NB)] + [
            pltpu.VMEM_SHARED((n_out, d), jnp.float32),
            pltpu.SemaphoreType.DMA,
            pltpu.SemaphoreType.DMA,
            pltpu.SemaphoreType.DMA,
            pltpu.SemaphoreType.DMA,
        ],
    )
    def segsum(hw_hbm, edges_hbm, out_hbm, sidx, didx,
               a0, a1, a2, a3, b0, b1, b2, b3, acc, gsa, gsb, ssa, ssb):
        c = lax.axis_index("c")
        s = lax.axis_index("s")
        wid = s * _NC + c
        bufa = [a0, a1, a2, a3]
        bufb = [b0, b1, b2, b3]

        zv = jnp.zeros((16,), jnp.float32)

        def zrow(i, _):
            for j in range(d // 16):
                a0[i, pl.ds(j * 16, 16)] = zv
            return 0

        lax.fori_loop(0, _CHUNK, zrow, 0)
        zbase = s * rpt
        nfull, rem = divmod(rpt, _CHUNK)
        for j in range(nfull):
            pltpu.sync_copy(a0, acc.at[pl.ds(zbase + j * _CHUNK, _CHUNK), :])
        if rem:
            pltpu.sync_copy(a0.at[pl.ds(0, rem), :],
                            acc.at[pl.ds(zbase + nfull * _CHUNK, rem), :])

        pltpu.sync_copy(edges_hbm.at[0, wid], sidx)
        pltpu.sync_copy(edges_hbm.at[1, wid], didx)
        plsc.subcore_barrier()

        def fire_g(j, buf, sem):
            return pltpu.async_copy(hw_hbm.at[sidx.at[j]], buf, sem)

        def fire_s(j, buf, sem):
            return pltpu.async_copy(buf, acc.at[didx.at[j]], sem, add=True)

        def half_round(base, bufs, gsem, ssem, nxt):
            hs = []
            for i in range(NB):
                pltpu.make_async_copy(hw_hbm.at[sidx.at[base + i]],
                                      bufs[i], gsem).wait()
                hs.append(fire_s(base + i, bufs[i], ssem))
            for h in hs:
                h.wait()
            if nxt is not None:
                for i in range(NB):
                    fire_g(nxt + i, bufs[i], gsem)

        for i in range(NB):
            fire_g(i, bufa[i], gsa)

        def round_body(r, _):
            base = 2 * NB * r
            for i in range(NB):
                fire_g(base + NB + i, bufb[i], gsb)
            half_round(base, bufa, gsa, ssa, base + 2 * NB)
            half_round(base + NB, bufb, gsb, ssb, None)
            return 0

        lax.fori_loop(0, R - 1, round_body, 0)
        base = 2 * NB * (R - 1)
        for i in range(NB):
            fire_g(base + NB + i, bufb[i], gsb)
        half_round(base, bufa, gsa, ssa, None)
        half_round(base + NB, bufb, gsb, ssb, None)

        plsc.subcore_barrier()
        pltpu.sync_copy(acc.at[pl.ds(s * rpt, rpt), :],
                        out_hbm.at[c, pl.ds(s * rpt, rpt), :])

    return segsum


def _first_body(x_ref, w_ref, c0_ref, c1_ref, hw_ref, dis_ref):
    dis = lax.rsqrt(1.0 + c0_ref[...] + c1_ref[...])
    dis_ref[...] = dis
    hw_ref[...] = jnp.dot(x_ref[...], w_ref[...],
                          preferred_element_type=jnp.float32) * dis


def _first_call(x, wt, c0, c1):
    n = x.shape[0]
    return pl.pallas_call(
        _first_body,
        out_shape=(jax.ShapeDtypeStruct((n, wt.shape[1]), jnp.float32),
                   jax.ShapeDtypeStruct((n, 1), jnp.float32)),
    )(x, wt, c0, c1)


def _mid_body(acc_ref, hwp_ref, dis_ref, b_ref, w_ref, out_ref):
    n = hwp_ref.shape[0]
    dis = dis_ref[...]
    h = (acc_ref[0, :n, :] + acc_ref[1, :n, :] + hwp_ref[...]) * dis + b_ref[...]
    h = jnp.maximum(h, 0.0)
    out_ref[...] = jnp.dot(h, w_ref[...],
                           preferred_element_type=jnp.float32) * dis


def _mid_call(acc, hwp, dis, b, wt):
    n = hwp.shape[0]
    return pl.pallas_call(
        _mid_body,
        out_shape=jax.ShapeDtypeStruct((n, wt.shape[1]), jnp.float32),
    )(acc, hwp, dis, b, wt)


def _fin_body(acc_ref, hwp_ref, dis_ref, b_ref, out_ref):
    n = hwp_ref.shape[0]
    out_ref[...] = (acc_ref[0, :n, :] + acc_ref[1, :n, :] + hwp_ref[...]) \
        * dis_ref[...] + b_ref[...]


def _fin_call(acc, hwp, dis, b):
    return pl.pallas_call(
        _fin_body,
        out_shape=jax.ShapeDtypeStruct(hwp.shape, jnp.float32),
    )(acc, hwp, dis, b)


def kernel(x, edge_index, W1, b1, W2, b2, W3, b3):
    n = x.shape[0]
    e = edge_index.shape[1]
    per = _NW * _CHUNK
    k = -(-(-(-e // per)) // 16) * 16
    pad = k * per - e

    pidx = jnp.arange(pad, dtype=jnp.int32)
    pad2d = jnp.stack([pidx % jnp.int32(min(_CHUNK, n)),
                       n + (pidx % jnp.int32(16))])
    et = jnp.concatenate([edge_index, pad2d], axis=1).reshape(2, _NW, k, _CHUNK)

    tile_rows = -(-(n + 16) // (_NS * 128)) * 128
    n_cnt = tile_rows * _NS

    counts = _make_count(k, n_cnt)(et)
    c0 = counts[:n].reshape(n, 1)
    c1 = counts[n_cnt:n_cnt + n].reshape(n, 1)

    hw1, dis = _first_call(x, W1.T, c0, c1)
    acc1 = _make_segsum(n, W1.shape[0], k)(hw1, et)
    hw2 = _mid_call(acc1, hw1, dis, b1.reshape(1, -1), W2.T)
    acc2 = _make_segsum(n, W2.shape[0], k)(hw2, et)
    hw3 = _mid_call(acc2, hw2, dis, b2.reshape(1, -1), W3.T)
    acc3 = _make_segsum(n, W3.shape[0], k)(hw3, et)
    z = _fin_call(acc3, hw3, dis, b3.reshape(1, -1))
    return z
